# baseline (speedup 1.0000x reference)
import functools

import jax
import jax.numpy as jnp
from jax import lax
from jax.experimental import pallas as pl
from jax.experimental.pallas import tpu as pltpu
from jax.experimental.pallas import tpu_sc as plsc

_POS_MIN = 0.7
_B, _C, _H, _W = 8, 1, 384, 384
_TOTAL = _B * _C * _H * _W
_ROWS = _B * _C * _H
_NC, _NS, _L = 2, 16, 16
_NW = _NC * _NS
_SC_ROWS = 1024
_ROWS_PER_W = _SC_ROWS // _NW
_CHR = 16
_NCHUNK = _ROWS_PER_W // _CHR
_CSTEPS = _W // _L
_TC_BLK = 128
_TC_STEPS = (_ROWS - _SC_ROWS) // _TC_BLK
_TC_OFF = _SC_ROWS // _TC_BLK


def _sc_partials(gr, pr, ga, pa, cm):
    mesh = plsc.VectorSubcoreMesh(core_axis_name="c", subcore_axis_name="s")

    @functools.partial(
        pl.kernel,
        out_type=jax.ShapeDtypeStruct((_NW, 6, _L), jnp.float32),
        mesh=mesh,
        compiler_params=pltpu.CompilerParams(use_tc_tiling_on_sc=True),
        scratch_types=[pltpu.VMEM((2, _CHR, _W), jnp.float32) for _ in range(5)]
        + [
            pltpu.VMEM((6, _L), jnp.float32),
            pltpu.SemaphoreType.DMA,
            pltpu.SemaphoreType.DMA,
        ],
    )
    def k(gr_h, pr_h, ga_h, pa_h, cm_h, out_h, bgr, bpr, bga, bpa, bcm,
          obuf, sem0, sem1):
        cid = lax.axis_index("c")
        sid = lax.axis_index("s")
        wid = cid * _NS + sid
        row0 = wid * _ROWS_PER_W
        hbm = (gr_h, pr_h, ga_h, pa_h, cm_h)
        bufs = (bgr, bpr, bga, bpa, bcm)
        sems = (sem0, sem1)

        def issue(c, slot):
            r0 = row0 + c * _CHR
            cps = []
            for h, bv in zip(hbm, bufs):
                cp = pltpu.make_async_copy(
                    h.at[pl.ds(r0, _CHR), :], bv.at[slot], sems[slot]
                )
                cp.start()
                cps.append(cp)
            return cps

        def compute(slot, accs):
            def row_body(r, a):
                def col_body(cstep, a2):
                    tsr, psr, pcr, tsa, psa, pca = a2
                    s = pl.ds(cstep * _L, _L)
                    g = bgr[slot, r, s]
                    p = bpr[slot, r, s]
                    h = bga[slot, r, s]
                    q = bpa[slot, r, s]
                    w = bcm[slot, r, s]
                    zero = jnp.zeros((_L,), jnp.float32)
                    one = jnp.full((_L,), 1.0, jnp.float32)
                    dr = g - p
                    lr = dr * dr * w
                    mr = g > _POS_MIN
                    da = h - q
                    la = da * da * w
                    ma = h > _POS_MIN
                    return (
                        tsr + lr,
                        psr + jnp.where(mr, lr, zero),
                        pcr + jnp.where(mr, one, zero),
                        tsa + la,
                        psa + jnp.where(ma, la, zero),
                        pca + jnp.where(ma, one, zero),
                    )

                return lax.fori_loop(0, _CSTEPS, col_body, a)

            return lax.fori_loop(0, _CHR, row_body, accs)

        z = jnp.zeros((_L,), jnp.float32)
        accs = (z, z, z, z, z, z)
        inflight = issue(0, 0)
        for c in range(_NCHUNK):
            slot = c % 2
            nxt = None
            if c + 1 < _NCHUNK:
                nxt = issue(c + 1, 1 - slot)
            for cp in inflight:
                cp.wait()
            accs = compute(slot, accs)
            inflight = nxt
        for j in range(6):
            obuf[j, :] = accs[j]
        pltpu.sync_copy(obuf, out_h.at[wid])

    return k(gr, pr, ga, pa, cm)


def _tc_partials(gr, pr, ga, pa, cm):
    def kern(g_ref, p_ref, h_ref, q_ref, w_ref, out_ref):
        i = pl.program_id(0)

        @pl.when(i == 0)
        def _():
            for j in range(8):
                out_ref[j] = jnp.float32(0.0)

        g = g_ref[...]
        p = p_ref[...]
        h = h_ref[...]
        q = q_ref[...]
        w = w_ref[...]
        dr = g - p
        lr = dr * dr * w
        mr = g > _POS_MIN
        da = h - q
        la = da * da * w
        ma = h > _POS_MIN
        out_ref[0] += jnp.sum(lr)
        out_ref[1] += jnp.sum(jnp.where(mr, lr, 0.0))
        out_ref[2] += jnp.sum(mr.astype(jnp.float32))
        out_ref[3] += jnp.sum(la)
        out_ref[4] += jnp.sum(jnp.where(ma, la, 0.0))
        out_ref[5] += jnp.sum(ma.astype(jnp.float32))

    bs = pl.BlockSpec((_TC_BLK, _W), lambda i: (i + _TC_OFF, 0))
    return pl.pallas_call(
        kern,
        grid=(_TC_STEPS,),
        out_shape=jax.ShapeDtypeStruct((8,), jnp.float32),
        in_specs=[bs] * 5,
        out_specs=pl.BlockSpec(memory_space=pltpu.SMEM),
    )(gr, pr, ga, pa, cm)


_RROWS = 9


def _hard_topk_sum(gt, pred, conf, kf):
    r3 = lambda a: jnp.reshape(a, (_RROWS, 128, 1024))
    gt3, pred3, conf3 = r3(gt), r3(pred), r3(conf)

    def kern(kf_ref, g_ref, p_ref, c_ref, out_ref, neg_ref):
        for j in range(_RROWS):
            g = g_ref[j]
            d = g - p_ref[j]
            l = d * d * c_ref[j]
            neg_ref[j] = jnp.where(g > _POS_MIN, 0.0, l)
        kf_ = kf_ref[0]

        def cnt_ge(t):
            def b(j, acc):
                return acc + jnp.sum((neg_ref[j] >= t).astype(jnp.float32))

            return lax.fori_loop(0, _RROWS, b, jnp.float32(0.0))

        def bs(_, lohi):
            lo, hi = lohi
            mid = (lo + hi) // 2
            t = lax.bitcast_convert_type(mid, jnp.float32)
            ok = cnt_ge(t) >= kf_
            return (jnp.where(ok, mid, lo), jnp.where(ok, hi, mid))

        lo, _ = lax.fori_loop(
            0, 31, bs, (jnp.int32(0), jnp.int32(0x3F800001))
        )
        t = lax.bitcast_convert_type(lo, jnp.float32)

        def b2(j, acc):
            s, c = acc
            v = neg_ref[j]
            m = v > t
            return (
                s + jnp.sum(jnp.where(m, v, 0.0)),
                c + jnp.sum(m.astype(jnp.float32)),
            )

        s, c = lax.fori_loop(0, _RROWS, b2, (jnp.float32(0.0), jnp.float32(0.0)))
        out_ref[0] = jnp.where(kf_ > 0.0, s + (kf_ - c) * t, 0.0)

    res = pl.pallas_call(
        kern,
        out_shape=jax.ShapeDtypeStruct((1,), jnp.float32),
        in_specs=[
            pl.BlockSpec(memory_space=pltpu.SMEM),
            pl.BlockSpec(memory_space=pltpu.VMEM),
            pl.BlockSpec(memory_space=pltpu.VMEM),
            pl.BlockSpec(memory_space=pltpu.VMEM),
        ],
        out_specs=pl.BlockSpec(memory_space=pltpu.SMEM),
        scratch_shapes=[pltpu.VMEM((_RROWS, 128, 1024), jnp.float32)],
    )(jnp.reshape(kf, (1,)), gt3, pred3, conf3)
    return res[0]


def _one_loss_any(ts, ps, pc, gt, pred, conf):
    total_f = jnp.float32(_TOTAL)

    def easy(_):
        return ts / total_f

    def hard(_):
        kf = 3.0 * pc
        topk = _hard_topk_sum(gt, pred, conf, kf)
        return (topk + ps) / (4.0 * pc)

    return lax.cond(4.0 * pc >= total_f, easy, hard, operand=None)


def kernel(gt_region, pred_region, gt_affinity, pred_affinity, conf_map):
    r2 = lambda a: jnp.reshape(a, (_ROWS, _W))
    gr, pr, ga, pa, cm = (
        r2(gt_region),
        r2(pred_region),
        r2(gt_affinity),
        r2(pred_affinity),
        r2(conf_map),
    )
    sc = _sc_partials(gr, pr, ga, pa, cm)
    tc = _tc_partials(gr, pr, ga, pa, cm)
    s = jnp.sum(sc, axis=(0, 2)) + tc[:6]
    total_f = jnp.float32(_TOTAL)
    both_easy = jnp.logical_and(
        4.0 * s[2] >= total_f, 4.0 * s[5] >= total_f
    )

    def easy(_):
        return (s[0] + s[3]) / total_f

    def hard(_):
        res_r = _one_loss_any(s[0], s[1], s[2], gr, pr, cm)
        res_a = _one_loss_any(s[3], s[4], s[5], ga, pa, cm)
        return res_r + res_a

    return lax.cond(both_easy, easy, hard, operand=None)


# speedup vs baseline: 1.1059x; 1.1059x over previous
import functools

import jax
import jax.numpy as jnp
from jax import lax
from jax.experimental import pallas as pl
from jax.experimental.pallas import tpu as pltpu
from jax.experimental.pallas import tpu_sc as plsc

_POS_MIN = 0.7
_B, _C, _H, _W = 8, 1, 384, 384
_TOTAL = _B * _C * _H * _W
_ROWS = _B * _C * _H
_NC, _NS, _L = 2, 16, 16
_NW = _NC * _NS
_SC_ROWS = 1024
_ROWS_PER_W = _SC_ROWS // _NW
_CHR = 16
_NCHUNK = _ROWS_PER_W // _CHR
_CSTEPS = _W // _L
_TC_BLK = 256
_TC_STEPS = (_ROWS - _SC_ROWS) // _TC_BLK
_TC_OFF = _SC_ROWS // _TC_BLK


def _sc_partials(gr, pr, ga, pa, cm):
    mesh = plsc.VectorSubcoreMesh(core_axis_name="c", subcore_axis_name="s")

    @functools.partial(
        pl.kernel,
        out_type=jax.ShapeDtypeStruct((_NW, 6, _L), jnp.float32),
        mesh=mesh,
        compiler_params=pltpu.CompilerParams(use_tc_tiling_on_sc=True),
        scratch_types=[pltpu.VMEM((2, _CHR, _W), jnp.float32) for _ in range(5)]
        + [
            pltpu.VMEM((6, _L), jnp.float32),
            pltpu.SemaphoreType.DMA,
            pltpu.SemaphoreType.DMA,
        ],
    )
    def k(gr_h, pr_h, ga_h, pa_h, cm_h, out_h, bgr, bpr, bga, bpa, bcm,
          obuf, sem0, sem1):
        cid = lax.axis_index("c")
        sid = lax.axis_index("s")
        wid = cid * _NS + sid
        row0 = wid * _ROWS_PER_W
        hbm = (gr_h, pr_h, ga_h, pa_h, cm_h)
        bufs = (bgr, bpr, bga, bpa, bcm)
        sems = (sem0, sem1)

        def issue(c, slot):
            r0 = row0 + c * _CHR
            cps = []
            for h, bv in zip(hbm, bufs):
                cp = pltpu.make_async_copy(
                    h.at[pl.ds(r0, _CHR), :], bv.at[slot], sems[slot]
                )
                cp.start()
                cps.append(cp)
            return cps

        def compute(slot, accs):
            def row_body(r, a):
                def col_body(cstep, a2):
                    tsr, psr, pcr, tsa, psa, pca = a2
                    s = pl.ds(cstep * _L, _L)
                    g = bgr[slot, r, s]
                    p = bpr[slot, r, s]
                    h = bga[slot, r, s]
                    q = bpa[slot, r, s]
                    w = bcm[slot, r, s]
                    zero = jnp.zeros((_L,), jnp.float32)
                    one = jnp.full((_L,), 1.0, jnp.float32)
                    dr = g - p
                    lr = dr * dr * w
                    mr = g > _POS_MIN
                    da = h - q
                    la = da * da * w
                    ma = h > _POS_MIN
                    return (
                        tsr + lr,
                        psr + jnp.where(mr, lr, zero),
                        pcr + jnp.where(mr, one, zero),
                        tsa + la,
                        psa + jnp.where(ma, la, zero),
                        pca + jnp.where(ma, one, zero),
                    )

                return lax.fori_loop(0, _CSTEPS, col_body, a)

            return lax.fori_loop(0, _CHR, row_body, accs)

        z = jnp.zeros((_L,), jnp.float32)
        accs = (z, z, z, z, z, z)
        inflight = issue(0, 0)
        for c in range(_NCHUNK):
            slot = c % 2
            nxt = None
            if c + 1 < _NCHUNK:
                nxt = issue(c + 1, 1 - slot)
            for cp in inflight:
                cp.wait()
            accs = compute(slot, accs)
            inflight = nxt
        for j in range(6):
            obuf[j, :] = accs[j]
        pltpu.sync_copy(obuf, out_h.at[wid])

    return k(gr, pr, ga, pa, cm)


def _tc_partials(gr, pr, ga, pa, cm):
    def kern(g_ref, p_ref, h_ref, q_ref, w_ref, out_ref):
        i = pl.program_id(0)

        @pl.when(i == 0)
        def _():
            for j in range(8):
                out_ref[j] = jnp.float32(0.0)

        g = g_ref[...]
        p = p_ref[...]
        h = h_ref[...]
        q = q_ref[...]
        w = w_ref[...]
        dr = g - p
        lr = dr * dr * w
        mr = g > _POS_MIN
        da = h - q
        la = da * da * w
        ma = h > _POS_MIN
        out_ref[0] += jnp.sum(lr)
        out_ref[1] += jnp.sum(jnp.where(mr, lr, 0.0))
        out_ref[2] += jnp.sum(mr.astype(jnp.float32))
        out_ref[3] += jnp.sum(la)
        out_ref[4] += jnp.sum(jnp.where(ma, la, 0.0))
        out_ref[5] += jnp.sum(ma.astype(jnp.float32))

    bs = pl.BlockSpec((_TC_BLK, _W), lambda i: (i + _TC_OFF, 0))
    return pl.pallas_call(
        kern,
        grid=(_TC_STEPS,),
        out_shape=jax.ShapeDtypeStruct((8,), jnp.float32),
        in_specs=[bs] * 5,
        out_specs=pl.BlockSpec(memory_space=pltpu.SMEM),
    )(gr, pr, ga, pa, cm)


_RROWS = 9


def _hard_topk_sum(gt, pred, conf, kf):
    r3 = lambda a: jnp.reshape(a, (_RROWS, 128, 1024))
    gt3, pred3, conf3 = r3(gt), r3(pred), r3(conf)

    def kern(kf_ref, g_ref, p_ref, c_ref, out_ref, neg_ref):
        for j in range(_RROWS):
            g = g_ref[j]
            d = g - p_ref[j]
            l = d * d * c_ref[j]
            neg_ref[j] = jnp.where(g > _POS_MIN, 0.0, l)
        kf_ = kf_ref[0]

        def cnt_ge(t):
            def b(j, acc):
                return acc + jnp.sum((neg_ref[j] >= t).astype(jnp.float32))

            return lax.fori_loop(0, _RROWS, b, jnp.float32(0.0))

        def bs(_, lohi):
            lo, hi = lohi
            mid = (lo + hi) // 2
            t = lax.bitcast_convert_type(mid, jnp.float32)
            ok = cnt_ge(t) >= kf_
            return (jnp.where(ok, mid, lo), jnp.where(ok, hi, mid))

        lo, _ = lax.fori_loop(
            0, 31, bs, (jnp.int32(0), jnp.int32(0x3F800001))
        )
        t = lax.bitcast_convert_type(lo, jnp.float32)

        def b2(j, acc):
            s, c = acc
            v = neg_ref[j]
            m = v > t
            return (
                s + jnp.sum(jnp.where(m, v, 0.0)),
                c + jnp.sum(m.astype(jnp.float32)),
            )

        s, c = lax.fori_loop(0, _RROWS, b2, (jnp.float32(0.0), jnp.float32(0.0)))
        out_ref[0] = jnp.where(kf_ > 0.0, s + (kf_ - c) * t, 0.0)

    res = pl.pallas_call(
        kern,
        out_shape=jax.ShapeDtypeStruct((1,), jnp.float32),
        in_specs=[
            pl.BlockSpec(memory_space=pltpu.SMEM),
            pl.BlockSpec(memory_space=pltpu.VMEM),
            pl.BlockSpec(memory_space=pltpu.VMEM),
            pl.BlockSpec(memory_space=pltpu.VMEM),
        ],
        out_specs=pl.BlockSpec(memory_space=pltpu.SMEM),
        scratch_shapes=[pltpu.VMEM((_RROWS, 128, 1024), jnp.float32)],
    )(jnp.reshape(kf, (1,)), gt3, pred3, conf3)
    return res[0]


def _one_loss_any(ts, ps, pc, gt, pred, conf):
    total_f = jnp.float32(_TOTAL)

    def easy(_):
        return ts / total_f

    def hard(_):
        kf = 3.0 * pc
        topk = _hard_topk_sum(gt, pred, conf, kf)
        return (topk + ps) / (4.0 * pc)

    return lax.cond(4.0 * pc >= total_f, easy, hard, operand=None)


def kernel(gt_region, pred_region, gt_affinity, pred_affinity, conf_map):
    r2 = lambda a: jnp.reshape(a, (_ROWS, _W))
    gr, pr, ga, pa, cm = (
        r2(gt_region),
        r2(pred_region),
        r2(gt_affinity),
        r2(pred_affinity),
        r2(conf_map),
    )
    sc = _sc_partials(gr, pr, ga, pa, cm)
    tc = _tc_partials(gr, pr, ga, pa, cm)
    s = jnp.sum(sc, axis=(0, 2)) + tc[:6]
    total_f = jnp.float32(_TOTAL)
    both_easy = jnp.logical_and(
        4.0 * s[2] >= total_f, 4.0 * s[5] >= total_f
    )

    def easy(_):
        return (s[0] + s[3]) / total_f

    def hard(_):
        res_r = _one_loss_any(s[0], s[1], s[2], gr, pr, cm)
        res_a = _one_loss_any(s[3], s[4], s[5], ga, pa, cm)
        return res_r + res_a

    return lax.cond(both_easy, easy, hard, operand=None)
